# Initial kernel scaffold; baseline (speedup 1.0000x reference)
import jax
import jax.numpy as jnp
from jax.experimental import pallas as pl


def kernel(local_entity, batch_heads, batch_rels, batch_tails, batch_ids, fact_ids, weight_list, weight_rel_list, rel_features, W, b):
    raise NotImplementedError("write your pallas kernel here")


# speedup vs baseline: 5.7904x; 5.7904x over previous
import functools

import jax
import jax.numpy as jnp
from jax import lax
from jax.experimental import pallas as pl
from jax.experimental.pallas import tpu as pltpu
from jax.experimental.pallas import tpu_sc as plsc

NC = 2
NS = 16
NT = NC * NS
KE = 128
LANES = 16


def _relval_body(rel_ref, w_ref, b_ref, out_ref):
    out_ref[...] = lax.dot_general(
        rel_ref[...], w_ref[...], (((1,), (1,)), ((), ())),
        preferred_element_type=jnp.float32) + b_ref[...]


def _addrelu_body(p_ref, o_ref):
    o_ref[...] = jnp.maximum(p_ref[0] + p_ref[1], 0.0)


@functools.lru_cache(maxsize=None)
def _make_sc_scatter(E, NENT, H):
    nblk = E // KE
    base, extra = nblk // NT, nblk % NT
    chunk = 200
    nchunk = NENT // chunk
    cbase, cextra = nchunk // NS, nchunk % NS
    mesh = plsc.VectorSubcoreMesh(core_axis_name="c", subcore_axis_name="s")

    @functools.partial(
        pl.kernel,
        out_type=jax.ShapeDtypeStruct((NC, NENT, H), jnp.float32),
        mesh=mesh,
        scratch_types=[
            pltpu.VMEM_SHARED((NENT, H), jnp.float32),
            pltpu.VMEM((KE,), jnp.int32),
            pltpu.VMEM((KE,), jnp.int32),
            pltpu.VMEM((KE,), jnp.int32),
            pltpu.VMEM((KE,), jnp.float32),
            pltpu.VMEM((KE, H), jnp.float32),
            pltpu.SemaphoreType.DMA,
        ],
    )
    def sc_scatter(relval_hbm, rels_hbm, tails_hbm, heads_hbm, w_hbm,
                   zeros_hbm, out_hbm, acc, ridx, tidx, hidx, wv, rows, sem):
        cid = lax.axis_index("c")
        sid = lax.axis_index("s")
        wid = sid * NC + cid
        nch = cbase + jnp.where(sid < cextra, 1, 0)

        def zero_body(j, carry):
            c = sid + NS * j
            pltpu.sync_copy(zeros_hbm, acc.at[pl.ds(c * chunk, chunk)])
            return carry

        lax.fori_loop(0, nch, zero_body, 0)
        plsc.subcore_barrier()

        nb = base + jnp.where(wid < extra, 1, 0)

        def block_body(j, carry):
            g = wid + NT * j
            pltpu.sync_copy(rels_hbm.at[g], ridx)
            pltpu.async_copy(relval_hbm.at[ridx], rows, sem).wait()
            pltpu.sync_copy(w_hbm.at[g], wv)
            pltpu.sync_copy(tails_hbm.at[g], tidx)
            pltpu.sync_copy(heads_hbm.at[g], hidx)

            def scale(grp, c2):
                wvec = wv[pl.ds(grp * LANES, LANES)]
                for i in range(LANES):
                    k = grp * LANES + i
                    wk = wvec[i]
                    for c in range(H // LANES):
                        sl = pl.ds(c * LANES, LANES)
                        rows[k, sl] = rows[k, sl] * wk
                return c2

            lax.fori_loop(0, KE // LANES, scale, 0)
            pltpu.sync_copy(rows, acc.at[tidx], add=True)
            pltpu.sync_copy(rows, acc.at[hidx], add=True)
            return carry

        lax.fori_loop(0, nb, block_body, 0)
        plsc.subcore_barrier()

        def out_body(j, carry):
            c = sid + NS * j
            sl = pl.ds(c * chunk, chunk)
            pltpu.sync_copy(acc.at[sl], out_hbm.at[cid, sl])
            return carry

        lax.fori_loop(0, nch, out_body, 0)

    return sc_scatter


def kernel(local_entity, batch_heads, batch_rels, batch_tails, batch_ids,
           fact_ids, weight_list, weight_rel_list, rel_features, W, b):
    bsz, max_local_entity = local_entity.shape
    nent = bsz * max_local_entity
    R, H = rel_features.shape
    E = batch_rels.shape[0]

    rel_val = pl.pallas_call(
        _relval_body,
        out_shape=jax.ShapeDtypeStruct((R, H), jnp.float32),
    )(rel_features, W, b.reshape(1, H))

    nblk = E // KE
    rels2 = batch_rels.astype(jnp.int32).reshape(nblk, KE)
    tails2 = batch_tails.astype(jnp.int32).reshape(nblk, KE)
    heads2 = batch_heads.astype(jnp.int32).reshape(nblk, KE)
    w2 = weight_rel_list.reshape(nblk, KE)
    zeros = jnp.zeros((200, H), jnp.float32)

    part = _make_sc_scatter(E, nent, H)(
        rel_val, rels2, tails2, heads2, w2, zeros)

    rows_blk = 2000
    out = pl.pallas_call(
        _addrelu_body,
        grid=(nent // rows_blk,),
        in_specs=[pl.BlockSpec((NC, rows_blk, H), lambda i: (0, i, 0))],
        out_specs=pl.BlockSpec((rows_blk, H), lambda i: (i, 0)),
        out_shape=jax.ShapeDtypeStruct((nent, H), jnp.float32),
    )(part)
    return out.reshape(bsz, max_local_entity, H)
